# baseline (speedup 1.0000x reference)
import functools

import jax
import jax.numpy as jnp
from jax import lax
from jax.experimental import pallas as pl
from jax.experimental.pallas import tpu as pltpu
from jax.experimental.pallas import tpu_sc as plsc

_TCH = 10


def _emb_kernel_factory(B, T, D, num_cores, num_subcores):
    nw = num_cores * num_subcores
    bpw = B // nw
    n_chunks = T // _TCH
    n_dt = D // 8
    n_bl = bpw // 16

    mesh = plsc.VectorSubcoreMesh(core_axis_name="c", subcore_axis_name="s")

    @functools.partial(
        pl.kernel,
        mesh=mesh,
        compiler_params=pltpu.CompilerParams(
            use_tc_tiling_on_sc=False, needs_layout_passes=False
        ),
        out_type=jax.ShapeDtypeStruct((T, D // 8, B // 128, 8, 128), jnp.float32),
        scratch_types=[
            pltpu.VMEM((T, bpw), jnp.int32),
            pltpu.VMEM((_TCH, bpw, D), jnp.float32),
            pltpu.VMEM((_TCH, D // 8, 8, 128), jnp.float32),
            pltpu.VMEM((T, D), jnp.float32),
            pltpu.SemaphoreType.DMA,
        ],
    )
    def emb_kernel(xT_hbm, tok_hbm, pos_hbm, out_hbm, idxT_v, grows_v, outv_v, pos_v, sem):
        wid = lax.axis_index("s") * num_cores + lax.axis_index("c")
        b0 = wid * bpw
        pltpu.sync_copy(pos_hbm, pos_v)
        pltpu.sync_copy(xT_hbm.at[:, pl.ds(b0, bpw)], idxT_v)

        lane = lax.iota(jnp.int32, 16)

        def chunk_body(c, carry):
            t0 = c * _TCH
            copies = [
                pltpu.async_copy(tok_hbm.at[idxT_v.at[t0 + tt]], grows_v.at[tt], sem)
                for tt in range(_TCH)
            ]
            for cp in copies:
                cp.wait()

            @plsc.parallel_loop(0, _TCH)
            def t_body(tt):
                stt = jnp.full((16,), tt, jnp.int32)
                st = t0 + stt
                for dt in range(n_dt):
                    for ds in range(8):
                        sd = jnp.full((16,), dt * 8 + ds, jnp.int32)
                        p = plsc.load_gather(pos_v, [st, sd])
                        for bl in range(n_bl):
                            v = plsc.load_gather(grows_v, [stt, lane + bl * 16, sd])
                            outv_v[tt, dt, ds, pl.ds(bl * 16, 16)] = v + p
            pltpu.sync_copy(outv_v, out_hbm.at[pl.ds(t0, _TCH), :, wid])
            return carry

        lax.fori_loop(0, n_chunks, chunk_body, 0)

    return emb_kernel


def kernel(x, token_table, pos_table):
    B, T = x.shape
    V, D = token_table.shape
    info = plsc.get_sparse_core_info()
    emb = _emb_kernel_factory(B, T, D, info.num_cores, info.num_subcores)
    out5 = emb(jnp.swapaxes(x, 0, 1).astype(jnp.int32), token_table, pos_table)
    return out5.transpose(2, 4, 0, 1, 3).reshape(B, T, D)


# speedup vs baseline: 1.1670x; 1.1670x over previous
import functools

import jax
import jax.numpy as jnp
from jax import lax
from jax.experimental import pallas as pl
from jax.experimental.pallas import tpu as pltpu
from jax.experimental.pallas import tpu_sc as plsc

_TCH = 10


def _emb_kernel_factory(B, T, D, num_cores, num_subcores):
    nw = num_cores * num_subcores
    bpw = B // nw
    n_chunks = T // _TCH
    n_dt = D // 8
    n_bl = bpw // 16

    mesh = plsc.VectorSubcoreMesh(core_axis_name="c", subcore_axis_name="s")

    @functools.partial(
        pl.kernel,
        mesh=mesh,
        compiler_params=pltpu.CompilerParams(
            use_tc_tiling_on_sc=False, needs_layout_passes=False
        ),
        out_type=jax.ShapeDtypeStruct((T, D // 8, B // 128, 8, 128), jnp.float32),
        scratch_types=[
            pltpu.VMEM((T, bpw), jnp.int32),
            pltpu.VMEM((_TCH, bpw, D), jnp.float32),
            pltpu.VMEM((_TCH, D // 8, 8, 128), jnp.float32),
            pltpu.VMEM((T, D), jnp.float32),
            pltpu.SemaphoreType.DMA,
        ],
    )
    def emb_kernel(xT_hbm, tok_hbm, pos_hbm, out_hbm, idxT_v, grows_v, outv_v, pos_v, sem):
        wid = lax.axis_index("s") * num_cores + lax.axis_index("c")
        b0 = wid * bpw
        pltpu.sync_copy(pos_hbm, pos_v)
        pltpu.sync_copy(xT_hbm.at[:, pl.ds(b0, bpw)], idxT_v)

        lane = lax.iota(jnp.int32, 16)

        def chunk_body(c, carry):
            t0 = c * _TCH
            copies = [
                pltpu.async_copy(tok_hbm.at[idxT_v.at[t0 + tt]], grows_v.at[tt], sem)
                for tt in range(_TCH)
            ]
            for cp in copies:
                cp.wait()

            @plsc.parallel_loop(0, _TCH)
            def t_body(tt):
                stt = jnp.full((16,), tt, jnp.int32)
                st = t0 + stt
                for dt in range(n_dt):
                    for ds in range(8):
                        sd = jnp.full((16,), dt * 8 + ds, jnp.int32)
                        p = plsc.load_gather(pos_v, [st, sd])
                        vs = [
                            plsc.load_gather(grows_v, [stt, lane + bl * 16, sd])
                            for bl in range(n_bl)
                        ]
                        for bl in range(n_bl):
                            outv_v[tt, dt, ds, pl.ds(bl * 16, 16)] = vs[bl] + p
            pltpu.sync_copy(outv_v, out_hbm.at[pl.ds(t0, _TCH), :, wid])
            return carry

        lax.fori_loop(0, n_chunks, chunk_body, 0)

    return emb_kernel


def kernel(x, token_table, pos_table):
    B, T = x.shape
    V, D = token_table.shape
    info = plsc.get_sparse_core_info()
    emb = _emb_kernel_factory(B, T, D, info.num_cores, info.num_subcores)
    out5 = emb(jnp.swapaxes(x, 0, 1).astype(jnp.int32), token_table, pos_table)
    return out5.transpose(2, 4, 0, 1, 3).reshape(B, T, D)


# speedup vs baseline: 1.9966x; 1.7108x over previous
import functools

import jax
import jax.numpy as jnp
from jax import lax
from jax.experimental import pallas as pl
from jax.experimental.pallas import tpu as pltpu
from jax.experimental.pallas import tpu_sc as plsc

_TCH = 10
_PAD = 129


def _emb_kernel_factory(B, T, D, num_cores, num_subcores):
    nw = num_cores * num_subcores
    bpw = B // nw
    n_chunks = T // _TCH
    n_dt = D // 8

    mesh = plsc.VectorSubcoreMesh(core_axis_name="c", subcore_axis_name="s")

    @functools.partial(
        pl.kernel,
        mesh=mesh,
        compiler_params=pltpu.CompilerParams(
            use_tc_tiling_on_sc=False, needs_layout_passes=False
        ),
        out_type=jax.ShapeDtypeStruct((T, D // 8, B // 128, 8, 128), jnp.float32),
        scratch_types=[
            pltpu.VMEM((T, bpw), jnp.int32),
            pltpu.VMEM((_TCH, bpw, D), jnp.float32),
            pltpu.VMEM((_TCH, D // 8, 8, _PAD), jnp.float32),
            pltpu.VMEM((T, D), jnp.float32),
            pltpu.SemaphoreType.DMA,
        ],
    )
    def emb_kernel(xT_hbm, tok_hbm, pos_hbm, out_hbm, idxT_v, grows_v, outv_v, pos_v, sem):
        wid = lax.axis_index("s") * num_cores + lax.axis_index("c")
        b0 = wid * bpw
        pltpu.sync_copy(pos_hbm, pos_v)
        pltpu.sync_copy(xT_hbm.at[:, pl.ds(b0, bpw)], idxT_v)

        lane = lax.iota(jnp.int32, 16)
        dt_idx = [lane // 8 + 2 * h for h in range(2)]
        ds_idx = lane % 8

        def chunk_body(c, carry):
            t0 = c * _TCH
            copies = [
                pltpu.async_copy(tok_hbm.at[idxT_v.at[t0 + tt]], grows_v.at[tt], sem)
                for tt in range(_TCH)
            ]
            for cp in copies:
                cp.wait()

            @plsc.parallel_loop(0, _TCH)
            def t_body(tt):
                stt = jnp.full((16,), tt, jnp.int32)
                t = t0 + tt
                ps = [pos_v[t, pl.ds(16 * h, 16)] for h in range(2)]

                @plsc.parallel_loop(0, bpw, unroll=4)
                def b_body(b):
                    sb = jnp.full((16,), b, jnp.int32)
                    for h in range(2):
                        v = grows_v[tt, b, pl.ds(16 * h, 16)] + ps[h]
                        plsc.store_scatter(
                            outv_v, [stt, dt_idx[h], ds_idx, sb], v
                        )

            pltpu.sync_copy(
                outv_v.at[:, :, :, pl.ds(0, 128)],
                out_hbm.at[pl.ds(t0, _TCH), :, wid],
            )
            return carry

        lax.fori_loop(0, n_chunks, chunk_body, 0)

    return emb_kernel


def kernel(x, token_table, pos_table):
    B, T = x.shape
    V, D = token_table.shape
    info = plsc.get_sparse_core_info()
    emb = _emb_kernel_factory(B, T, D, info.num_cores, info.num_subcores)
    out5 = emb(jnp.swapaxes(x, 0, 1).astype(jnp.int32), token_table, pos_table)
    return out5.transpose(2, 4, 0, 1, 3).reshape(B, T, D)


# speedup vs baseline: 2.1451x; 1.0744x over previous
import functools

import jax
import jax.numpy as jnp
from jax import lax
from jax.experimental import pallas as pl
from jax.experimental.pallas import tpu as pltpu
from jax.experimental.pallas import tpu_sc as plsc

_TCH = 5
_PAD = 129


def _emb_kernel_factory(B, T, D, num_cores, num_subcores):
    nw = num_cores * num_subcores
    bpw = B // nw
    n_chunks = T // _TCH
    n_dt = D // 8

    mesh = plsc.VectorSubcoreMesh(core_axis_name="c", subcore_axis_name="s")

    @functools.partial(
        pl.kernel,
        mesh=mesh,
        compiler_params=pltpu.CompilerParams(
            use_tc_tiling_on_sc=False, needs_layout_passes=False
        ),
        out_type=jax.ShapeDtypeStruct((T, D // 8, B // 128, 8, 128), jnp.float32),
        scratch_types=[
            pltpu.VMEM((T, bpw), jnp.int32),
            pltpu.VMEM((2, _TCH, bpw, D), jnp.float32),
            pltpu.VMEM((2, _TCH, D // 8, 8, _PAD), jnp.float32),
            pltpu.VMEM((T, D), jnp.float32),
            pltpu.SemaphoreType.DMA,
            pltpu.SemaphoreType.DMA,
            pltpu.SemaphoreType.DMA,
            pltpu.SemaphoreType.DMA,
        ],
    )
    def emb_kernel(
        xT_hbm, tok_hbm, pos_hbm, out_hbm,
        idxT_v, grows_v, outv_v, pos_v, sga, sgb, soa, sob,
    ):
        wid = lax.axis_index("s") * num_cores + lax.axis_index("c")
        b0 = wid * bpw
        pltpu.sync_copy(pos_hbm, pos_v)
        pltpu.sync_copy(xT_hbm.at[:, pl.ds(b0, bpw)], idxT_v)

        lane = lax.iota(jnp.int32, 16)
        dt_idx = [lane // 8 + 2 * h for h in range(2)]
        ds_idx = lane % 8
        sems_g = (sga, sgb)
        sems_o = (soa, sob)

        def fire(g):
            buf = g % 2
            return [
                pltpu.async_copy(
                    tok_hbm.at[idxT_v.at[g * _TCH + tt]],
                    grows_v.at[buf, tt],
                    sems_g[buf],
                )
                for tt in range(_TCH)
            ]

        def compute(g):
            buf = g % 2
            t0 = g * _TCH

            @plsc.parallel_loop(0, _TCH)
            def t_body(tt):
                stt = jnp.full((16,), tt, jnp.int32)
                t = t0 + tt
                ps = [pos_v[t, pl.ds(16 * h, 16)] for h in range(2)]

                @plsc.parallel_loop(0, bpw, unroll=4)
                def b_body(b):
                    sb = jnp.full((16,), b, jnp.int32)
                    for h in range(2):
                        v = grows_v[buf, tt, b, pl.ds(16 * h, 16)] + ps[h]
                        plsc.store_scatter(
                            outv_v.at[buf], [stt, dt_idx[h], ds_idx, sb], v
                        )

        def fire_out(g):
            buf = g % 2
            return pltpu.async_copy(
                outv_v.at[buf, :, :, :, pl.ds(0, 128)],
                out_hbm.at[pl.ds(g * _TCH, _TCH), :, wid],
                sems_o[buf],
            )

        pend_g = {0: fire(0)}
        pend_o = {}
        for g in range(n_chunks):
            buf = g % 2
            if g + 1 < n_chunks:
                pend_g[(g + 1) % 2] = fire(g + 1)
            for cp in pend_g[buf]:
                cp.wait()
            if g >= 2:
                pend_o[buf].wait()
            compute(g)
            pend_o[buf] = fire_out(g)
        pend_o[(n_chunks - 2) % 2].wait()
        pend_o[(n_chunks - 1) % 2].wait()

    return emb_kernel


def kernel(x, token_table, pos_table):
    B, T = x.shape
    V, D = token_table.shape
    info = plsc.get_sparse_core_info()
    emb = _emb_kernel_factory(B, T, D, info.num_cores, info.num_subcores)
    out5 = emb(jnp.swapaxes(x, 0, 1).astype(jnp.int32), token_table, pos_table)
    return out5.transpose(2, 4, 0, 1, 3).reshape(B, T, D)
